# baseline (speedup 1.0000x reference)
import functools

import jax
import jax.numpy as jnp
from jax import lax
from jax.experimental import pallas as pl
from jax.experimental.pallas import tpu as pltpu
from jax.experimental.pallas import tpu_sc as plsc

NC = 2
NS = 16
NW = NC * NS
LANES = 16
CHUNK = 128


def _encoder_body(tid_hbm, gid_hbm, ttab_hbm, gtab_hbm, te_hbm, out_hbm,
                  tidx_v, gidx_v, te_v, tbuf, gbuf, cbuf, tsem, gsem):
  cpw = tid_hbm.shape[0] // NW
  dim = ttab_hbm.shape[1]
  wid = lax.axis_index("s") * NC + lax.axis_index("c")
  base_chunk = wid * cpw

  pltpu.sync_copy(tid_hbm.at[pl.ds(base_chunk, cpw)], tidx_v)
  pltpu.sync_copy(gid_hbm.at[pl.ds(base_chunk, cpw)], gidx_v)
  pltpu.sync_copy(te_hbm, te_v)

  tc = [te_v[pl.ds(c * LANES, LANES)] for c in range(dim // LANES)]
  gc = [te_v[pl.ds(dim + c * LANES, LANES)] for c in range(dim // LANES)]

  for j in range(cpw):
    pltpu.async_copy(ttab_hbm.at[tidx_v.at[j]], tbuf, tsem).wait()
    pltpu.async_copy(gtab_hbm.at[gidx_v.at[j]], gbuf, gsem).wait()

    @pl.loop(0, CHUNK)
    def _row(i):
      for c in range(dim // LANES):
        sl = pl.ds(c * LANES, LANES)
        cbuf[i, pl.ds(c * LANES, LANES)] = tbuf[i, sl] + tc[c]
        cbuf[i, pl.ds(dim + c * LANES, LANES)] = gbuf[i, sl] + gc[c]

    pltpu.sync_copy(cbuf, out_hbm.at[pl.ds((base_chunk + j) * CHUNK, CHUNK)])


def kernel(text_id, goal_type_id, text_table, goal_table, type_embed):
  batch = text_id.shape[0]
  dim = text_table.shape[1]
  n_chunks = batch // CHUNK

  mesh = plsc.VectorSubcoreMesh(
      core_axis_name="c", subcore_axis_name="s",
      num_cores=NC, num_subcores=NS)

  run = functools.partial(
      pl.kernel,
      out_type=jax.ShapeDtypeStruct((batch, 2 * dim), jnp.float32),
      mesh=mesh,
      scratch_types=[
          pltpu.VMEM((n_chunks // NW, CHUNK), jnp.int32),
          pltpu.VMEM((n_chunks // NW, CHUNK), jnp.int32),
          pltpu.VMEM((2 * dim,), jnp.float32),
          pltpu.VMEM((CHUNK, dim), jnp.float32),
          pltpu.VMEM((CHUNK, dim), jnp.float32),
          pltpu.VMEM((CHUNK, 2 * dim), jnp.float32),
          pltpu.SemaphoreType.DMA,
          pltpu.SemaphoreType.DMA,
      ],
      compiler_params=pltpu.CompilerParams(use_tc_tiling_on_sc=False),
  )(_encoder_body)

  out = run(
      text_id.reshape(n_chunks, CHUNK),
      goal_type_id.reshape(n_chunks, CHUNK),
      text_table,
      goal_table,
      type_embed.reshape(2 * dim),
  )
  return out.reshape(batch, 2, dim)


# speedup vs baseline: 1.0196x; 1.0196x over previous
import functools

import jax
import jax.numpy as jnp
from jax import lax
from jax.experimental import pallas as pl
from jax.experimental.pallas import tpu as pltpu
from jax.experimental.pallas import tpu_sc as plsc

NC = 2
NS = 16
NW = NC * NS
LANES = 16
CHUNK = 128
NBUF = 2


def _encoder_body(tid_hbm, gid_hbm, ttab_hbm, gtab_hbm, te_hbm, out_hbm,
                  tidx_v, gidx_v, te_v, tbuf, gbuf, cbuf,
                  tsems, gsems, wsems):
  cpw = tid_hbm.shape[0] // NW
  dim = ttab_hbm.shape[1]
  ncs = dim // LANES
  wid = lax.axis_index("s") * NC + lax.axis_index("c")
  base_chunk = wid * cpw

  pltpu.sync_copy(tid_hbm.at[pl.ds(base_chunk, cpw)], tidx_v)
  pltpu.sync_copy(gid_hbm.at[pl.ds(base_chunk, cpw)], gidx_v)
  pltpu.sync_copy(te_hbm, te_v)

  tc = [te_v[pl.ds(c * LANES, LANES)] for c in range(ncs)]
  gc = [te_v[pl.ds(dim + c * LANES, LANES)] for c in range(ncs)]

  def issue_gathers(j):
    b = j % NBUF
    tcp = pltpu.make_async_copy(ttab_hbm.at[tidx_v.at[j]], tbuf.at[b],
                                tsems[b])
    gcp = pltpu.make_async_copy(gtab_hbm.at[gidx_v.at[j]], gbuf.at[b],
                                gsems[b])
    tcp.start()
    gcp.start()
    return tcp, gcp

  pending = {0: issue_gathers(0)}
  writes = {}
  for j in range(cpw):
    b = j % NBUF
    if j + 1 < cpw:
      pending[j + 1] = issue_gathers(j + 1)
    tcp, gcp = pending[j]
    tcp.wait()
    gcp.wait()
    if j >= NBUF:
      writes[j - NBUF].wait()

    @plsc.parallel_loop(0, CHUNK, unroll=4)
    def _row(i):
      for c in range(ncs):
        sl = pl.ds(c * LANES, LANES)
        cbuf[b, i, pl.ds(c * LANES, LANES)] = tbuf[b, i, sl] + tc[c]
        cbuf[b, i, pl.ds(dim + c * LANES, LANES)] = gbuf[b, i, sl] + gc[c]

    wcp = pltpu.make_async_copy(
        cbuf.at[b], out_hbm.at[pl.ds((base_chunk + j) * CHUNK, CHUNK)],
        wsems[b])
    wcp.start()
    writes[j] = wcp
  for j in range(max(0, cpw - NBUF), cpw):
    writes[j].wait()


def kernel(text_id, goal_type_id, text_table, goal_table, type_embed):
  batch = text_id.shape[0]
  dim = text_table.shape[1]
  n_chunks = batch // CHUNK

  mesh = plsc.VectorSubcoreMesh(
      core_axis_name="c", subcore_axis_name="s",
      num_cores=NC, num_subcores=NS)

  run = functools.partial(
      pl.kernel,
      out_type=jax.ShapeDtypeStruct((batch, 2 * dim), jnp.float32),
      mesh=mesh,
      scratch_types=[
          pltpu.VMEM((n_chunks // NW, CHUNK), jnp.int32),
          pltpu.VMEM((n_chunks // NW, CHUNK), jnp.int32),
          pltpu.VMEM((2 * dim,), jnp.float32),
          pltpu.VMEM((NBUF, CHUNK, dim), jnp.float32),
          pltpu.VMEM((NBUF, CHUNK, dim), jnp.float32),
          pltpu.VMEM((NBUF, CHUNK, 2 * dim), jnp.float32),
          [pltpu.SemaphoreType.DMA] * NBUF,
          [pltpu.SemaphoreType.DMA] * NBUF,
          [pltpu.SemaphoreType.DMA] * NBUF,
      ],
      compiler_params=pltpu.CompilerParams(use_tc_tiling_on_sc=False),
  )(_encoder_body)

  out = run(
      text_id.reshape(n_chunks, CHUNK),
      goal_type_id.reshape(n_chunks, CHUNK),
      text_table,
      goal_table,
      type_embed.reshape(2 * dim),
  )
  return out.reshape(batch, 2, dim)


# speedup vs baseline: 1.4714x; 1.4431x over previous
import functools

import jax
import jax.numpy as jnp
from jax import lax
from jax.experimental import pallas as pl
from jax.experimental.pallas import tpu as pltpu
from jax.experimental.pallas import tpu_sc as plsc

NC = 2
NS = 16
NW = NC * NS
LANES = 16
CHUNK = 32


def _encoder_body(tid_hbm, gid_hbm, ttab_hbm, gtab_hbm, te_hbm, out_hbm,
                  tidx_v, gidx_v, te_v, ttiles, gtiles, cbuf,
                  tsem, gsem, wsem):
  rows_pw = tid_hbm.shape[0] // NW
  cpw = rows_pw // CHUNK
  dim = ttab_hbm.shape[1]
  ncs = dim // LANES
  wid = lax.axis_index("s") * NC + lax.axis_index("c")

  pltpu.sync_copy(tid_hbm.at[pl.ds(wid * rows_pw, rows_pw)], tidx_v)
  pltpu.sync_copy(gid_hbm.at[pl.ds(wid * rows_pw, rows_pw)], gidx_v)
  pltpu.sync_copy(te_hbm, te_v)

  tc = [te_v[pl.ds(c * LANES, LANES)] for c in range(ncs)]
  gc = [te_v[pl.ds(dim + c * LANES, LANES)] for c in range(ncs)]

  @pl.loop(0, cpw)
  def _chunk(j):
    copies = []
    subs = {}
    for g in range(CHUNK // LANES):
      tv = tidx_v[pl.ds(j * CHUNK + g * LANES, LANES)]
      gv = gidx_v[pl.ds(j * CHUNK + g * LANES, LANES)]
      subs[g] = (tv & 7, gv & 7)
      tt = lax.shift_right_logical(tv, 3) * 8
      gt = lax.shift_right_logical(gv, 3) * 8
      for k in range(LANES):
        slot = g * LANES + k
        cp = pltpu.make_async_copy(
            ttab_hbm.at[pl.ds(pl.multiple_of(tt[k], 8), 8)],
            ttiles.at[slot], tsem)
        cp.start()
        copies.append(cp)
        cp = pltpu.make_async_copy(
            gtab_hbm.at[pl.ds(pl.multiple_of(gt[k], 8), 8)],
            gtiles.at[slot], gsem)
        cp.start()
        copies.append(cp)
    for cp in copies:
      cp.wait()

    @pl.when(j > 0)
    def _():
      pltpu.make_async_copy(
          cbuf, out_hbm.at[pl.ds(0, CHUNK)], wsem).wait()

    for g in range(CHUNK // LANES):
      trs, grs = subs[g]
      for k in range(LANES):
        row = g * LANES + k
        tr = trs[k]
        gr = grs[k]
        for c in range(ncs):
          sl = pl.ds(c * LANES, LANES)
          cbuf[row, pl.ds(c * LANES, LANES)] = ttiles[row, tr, sl] + tc[c]
          cbuf[row, pl.ds(dim + c * LANES, LANES)] = (
              gtiles[row, gr, sl] + gc[c])

    pltpu.make_async_copy(
        cbuf, out_hbm.at[pl.ds(wid * rows_pw + j * CHUNK, CHUNK)],
        wsem).start()

  pltpu.make_async_copy(cbuf, out_hbm.at[pl.ds(0, CHUNK)], wsem).wait()


def kernel(text_id, goal_type_id, text_table, goal_table, type_embed):
  batch = text_id.shape[0]
  vocab, dim = text_table.shape
  rows_pw = batch // NW

  mesh = plsc.VectorSubcoreMesh(
      core_axis_name="c", subcore_axis_name="s",
      num_cores=NC, num_subcores=NS)

  run = functools.partial(
      pl.kernel,
      out_type=jax.ShapeDtypeStruct((batch, 2 * dim), jnp.float32),
      mesh=mesh,
      scratch_types=[
          pltpu.VMEM((rows_pw,), jnp.int32),
          pltpu.VMEM((rows_pw,), jnp.int32),
          pltpu.VMEM((2 * dim,), jnp.float32),
          pltpu.VMEM((CHUNK, 8, dim), jnp.float32),
          pltpu.VMEM((CHUNK, 8, dim), jnp.float32),
          pltpu.VMEM((CHUNK, 2 * dim), jnp.float32),
          pltpu.SemaphoreType.DMA,
          pltpu.SemaphoreType.DMA,
          pltpu.SemaphoreType.DMA,
      ],
  )(_encoder_body)

  out = run(
      text_id,
      goal_type_id,
      text_table,
      goal_table,
      type_embed.reshape(2 * dim),
  )
  return out.reshape(batch, 2, dim)
